# baseline (speedup 1.0000x reference)
import functools

import jax
import jax.numpy as jnp
from jax.experimental import pallas as pl
from jax.experimental.pallas import tpu as pltpu

KNN = 16
LANES = 128


def _prep_body(xcn_ref, xnc_ref, wemb_ref, wattn_ref, beb_col_ref, beb_row_ref,
               battn_ref, xn_cn_ref, xn_nc_ref, sq_n_ref, sq_t_ref,
               s1_t_ref, s2_n_ref, xp_nc_ref):
    C = xcn_ref.shape[1]
    N = xcn_ref.shape[2]
    xb_cn = xcn_ref[0]
    xb_nc = xnc_ref[0]
    nrm_col = jnp.sqrt(jnp.sum(xb_cn * xb_cn, axis=1, keepdims=True))
    inv_col = 1.0 / jnp.maximum(nrm_col, 1e-12)
    xn_cn = xb_cn * inv_col
    xn_cn_ref[0, :, :N] = xn_cn
    xn_cn_ref[0, :, N:] = jnp.zeros_like(xn_cn_ref[0, :, N:])
    nrm_row = jnp.sqrt(jnp.sum(xb_nc * xb_nc, axis=0, keepdims=True))
    inv_row = 1.0 / jnp.maximum(nrm_row, 1e-12)
    xn_nc = xb_nc * inv_row
    xn_nc_ref[0] = xn_nc
    sq_n_ref[0, :, :N] = jnp.sum(xn_cn * xn_cn, axis=0, keepdims=True)
    sq_n_ref[0, :, N:] = jnp.full_like(sq_n_ref[0, :, N:], 1e30)
    sq_t_ref[0] = jnp.sum(xn_nc * xn_nc, axis=1, keepdims=True)
    wemb = wemb_ref[...]
    wattn = wattn_ref[...]
    a1 = wattn[:, :C]
    a2 = wattn[:, C:]
    e_nc = jax.lax.dot_general(xb_nc, wemb, (((1,), (1,)), ((), ())),
                               preferred_element_type=jnp.float32)
    e_nc = e_nc + beb_row_ref[...]
    s1_t_ref[0] = jax.lax.dot_general(e_nc, a1, (((1,), (1,)), ((), ())),
                                      preferred_element_type=jnp.float32)
    e_cn = jnp.dot(wemb, xb_cn, preferred_element_type=jnp.float32)
    e_cn = e_cn + beb_col_ref[...]
    s2 = jax.lax.dot_general(a2, e_cn, (((1,), (0,)), ((), ())),
                             preferred_element_type=jnp.float32)
    s2_n_ref[0, :, :N] = s2 + battn_ref[0, 0]
    s2_n_ref[0, :, N:] = jnp.zeros_like(s2_n_ref[0, :, N:])
    xp_nc_ref[0, :N, :] = xb_nc
    xp_nc_ref[0, N:, :] = jnp.zeros_like(xp_nc_ref[0, N:, :])


def _main_body(xn_nc_ref, xn_cn_ref, x_nc_ref, sq_n_ref, sq_t_ref,
               s1_t_ref, s2_n_ref, wct_ref, bconv_ref,
               y_ref, sums_ref, sumsq_ref, u_scr, *, blk_r, n_pad):
    b = pl.program_id(0)
    j = pl.program_id(1)
    C = xn_nc_ref.shape[2]

    xr = xn_nc_ref[0]
    xc = xn_cn_ref[0]
    d = sq_t_ref[0] + sq_n_ref[0] - 2.0 * jnp.dot(
        xr, xc, preferred_element_type=jnp.float32)
    u_scr[...] = jax.lax.bitcast_convert_type(jnp.maximum(d, 0.0), jnp.int32)

    nch = n_pad // LANES
    hibit = jnp.int32(-2**31)

    def step(_, m):
        mp1 = m + 1
        mp1b = mp1 ^ hibit
        chunks = [u_scr[:, i * LANES:(i + 1) * LANES] - mp1b
                  for i in range(nch)]
        while len(chunks) > 1:
            nxt = [jnp.minimum(chunks[2 * i], chunks[2 * i + 1])
                   for i in range(len(chunks) // 2)]
            if len(chunks) % 2:
                nxt.append(chunks[-1])
            chunks = nxt
        r = jnp.min(chunks[0], axis=1, keepdims=True)
        return mp1 + (r ^ hibit)

    m0 = jnp.full((blk_r, 1), -1, dtype=jnp.int32)
    t = jax.lax.fori_loop(0, KNN, step, m0)

    member = u_scr[...] <= t

    logit = s1_t_ref[0] + s2_n_ref[0]
    logit = jnp.where(logit >= 0, logit, 0.1 * logit)
    lm = jnp.where(member, logit, jnp.float32(-1e30))
    mx = jnp.max(lm, axis=1, keepdims=True)
    p = jnp.exp(lm - mx)
    w = p * (1.0 / jnp.sum(p, axis=1, keepdims=True))

    x_full = x_nc_ref[0]
    agg = jnp.dot(w, x_full, preferred_element_type=jnp.float32)
    x_rows = x_nc_ref[0, pl.ds(j * blk_r, blk_r), :]

    wct = wct_ref[...]
    y = (jnp.dot(x_rows, wct[:C], preferred_element_type=jnp.float32)
         + jnp.dot(agg, wct[C:], preferred_element_type=jnp.float32)
         + bconv_ref[...])
    y_ref[0] = y

    @pl.when(jnp.logical_and(b == 0, j == 0))
    def _():
        sums_ref[...] = jnp.zeros_like(sums_ref)
        sumsq_ref[...] = jnp.zeros_like(sumsq_ref)

    sums_ref[...] += jnp.sum(y, axis=0, keepdims=True)
    sumsq_ref[...] += jnp.sum(y * y, axis=0, keepdims=True)


def _final_body(y_ref, sums_ref, sumsq_ref, gamma_ref, beta_ref, x_nc_ref,
                out_ref, *, count):
    mean = sums_ref[...] / count
    var = sumsq_ref[...] / count - mean * mean
    inv = jax.lax.rsqrt(var + 1e-5)
    y = y_ref[0]
    z = gamma_ref[...] * (y - mean) * inv + beta_ref[...]
    z = jnp.maximum(z, 0.0)
    out_ref[0] = z + x_nc_ref[0]


def kernel(x, W_emb, b_emb, W_attn, b_attn, W_conv, b_conv, gamma, beta):
    B, C, H, W = x.shape
    N = H * W
    NP = ((N + LANES - 1) // LANES) * LANES
    x_cn = x.reshape(B, C, N)
    x_nc = x_cn.transpose(0, 2, 1)

    f32 = jnp.float32
    prep_out = pl.pallas_call(
        _prep_body,
        grid=(B,),
        in_specs=[
            pl.BlockSpec((1, C, N), lambda b: (b, 0, 0)),
            pl.BlockSpec((1, N, C), lambda b: (b, 0, 0)),
            pl.BlockSpec((C, C), lambda b: (0, 0)),
            pl.BlockSpec((1, 2 * C), lambda b: (0, 0)),
            pl.BlockSpec((C, 1), lambda b: (0, 0)),
            pl.BlockSpec((1, C), lambda b: (0, 0)),
            pl.BlockSpec((1, 1), lambda b: (0, 0)),
        ],
        out_specs=[
            pl.BlockSpec((1, C, NP), lambda b: (b, 0, 0)),
            pl.BlockSpec((1, N, C), lambda b: (b, 0, 0)),
            pl.BlockSpec((1, 1, NP), lambda b: (b, 0, 0)),
            pl.BlockSpec((1, N, 1), lambda b: (b, 0, 0)),
            pl.BlockSpec((1, N, 1), lambda b: (b, 0, 0)),
            pl.BlockSpec((1, 1, NP), lambda b: (b, 0, 0)),
            pl.BlockSpec((1, NP, C), lambda b: (b, 0, 0)),
        ],
        out_shape=[
            jax.ShapeDtypeStruct((B, C, NP), f32),
            jax.ShapeDtypeStruct((B, N, C), f32),
            jax.ShapeDtypeStruct((B, 1, NP), f32),
            jax.ShapeDtypeStruct((B, N, 1), f32),
            jax.ShapeDtypeStruct((B, N, 1), f32),
            jax.ShapeDtypeStruct((B, 1, NP), f32),
            jax.ShapeDtypeStruct((B, NP, C), f32),
        ],
    )(x_cn, x_nc, W_emb, W_attn, b_emb.reshape(C, 1), b_emb.reshape(1, C),
      b_attn.reshape(1, 1))
    xn_cn, xn_nc, sq_n, sq_t, s1_t, s2_n, xp_nc = prep_out

    blk_r = 448
    nb = N // blk_r
    y_nc, sums, sumsq = pl.pallas_call(
        functools.partial(_main_body, blk_r=blk_r, n_pad=NP),
        grid=(B, nb),
        in_specs=[
            pl.BlockSpec((1, blk_r, C), lambda b, j: (b, j, 0)),
            pl.BlockSpec((1, C, NP), lambda b, j: (b, 0, 0)),
            pl.BlockSpec((1, NP, C), lambda b, j: (b, 0, 0)),
            pl.BlockSpec((1, 1, NP), lambda b, j: (b, 0, 0)),
            pl.BlockSpec((1, blk_r, 1), lambda b, j: (b, j, 0)),
            pl.BlockSpec((1, blk_r, 1), lambda b, j: (b, j, 0)),
            pl.BlockSpec((1, 1, NP), lambda b, j: (b, 0, 0)),
            pl.BlockSpec((2 * C, C), lambda b, j: (0, 0)),
            pl.BlockSpec((1, C), lambda b, j: (0, 0)),
        ],
        out_specs=[
            pl.BlockSpec((1, blk_r, C), lambda b, j: (b, j, 0)),
            pl.BlockSpec((1, C), lambda b, j: (0, 0)),
            pl.BlockSpec((1, C), lambda b, j: (0, 0)),
        ],
        out_shape=[
            jax.ShapeDtypeStruct((B, N, C), f32),
            jax.ShapeDtypeStruct((1, C), f32),
            jax.ShapeDtypeStruct((1, C), f32),
        ],
        scratch_shapes=[pltpu.VMEM((blk_r, NP), jnp.int32)],
    )(xn_nc, xn_cn, xp_nc, sq_n, sq_t, s1_t, s2_n, W_conv.T,
      b_conv.reshape(1, C))

    out_nc = pl.pallas_call(
        functools.partial(_final_body, count=float(B * N)),
        grid=(B,),
        in_specs=[
            pl.BlockSpec((1, N, C), lambda b: (b, 0, 0)),
            pl.BlockSpec((1, C), lambda b: (0, 0)),
            pl.BlockSpec((1, C), lambda b: (0, 0)),
            pl.BlockSpec((1, C), lambda b: (0, 0)),
            pl.BlockSpec((1, C), lambda b: (0, 0)),
            pl.BlockSpec((1, N, C), lambda b: (b, 0, 0)),
        ],
        out_specs=pl.BlockSpec((1, N, C), lambda b: (b, 0, 0)),
        out_shape=jax.ShapeDtypeStruct((B, N, C), f32),
    )(y_nc, sums, sumsq, gamma.reshape(1, C), beta.reshape(1, C), x_nc)

    return out_nc.transpose(0, 2, 1).reshape(B, C, H, W)


# speedup vs baseline: 1.1546x; 1.1546x over previous
import functools

import jax
import jax.numpy as jnp
from jax.experimental import pallas as pl
from jax.experimental.pallas import tpu as pltpu

KNN = 16
LANES = 128


def _prep_body(xcn_ref, xnc_ref, wemb_ref, wattn_ref, beb_col_ref, beb_row_ref,
               battn_ref, xn_cn_ref, xn_nc_ref, sq_n_ref,
               s1_t_ref, s2_n_ref, xp_nc_ref):
    C = xcn_ref.shape[1]
    N = xcn_ref.shape[2]
    xb_cn = xcn_ref[0]
    xb_nc = xnc_ref[0]
    nrm_col = jnp.sqrt(jnp.sum(xb_cn * xb_cn, axis=1, keepdims=True))
    inv_col = 1.0 / jnp.maximum(nrm_col, 1e-12)
    xn_cn = xb_cn * inv_col
    xn_cn_ref[0, :, :N] = xn_cn
    xn_cn_ref[0, :, N:] = jnp.zeros_like(xn_cn_ref[0, :, N:])
    nrm_row = jnp.sqrt(jnp.sum(xb_nc * xb_nc, axis=0, keepdims=True))
    inv_row = 1.0 / jnp.maximum(nrm_row, 1e-12)
    xn_nc = xb_nc * inv_row
    xn_nc_ref[0] = xn_nc
    sq_n_ref[0, :, :N] = jnp.sum(xn_cn * xn_cn, axis=0, keepdims=True)
    sq_n_ref[0, :, N:] = jnp.full_like(sq_n_ref[0, :, N:], 1e30)
    wemb = wemb_ref[...]
    wattn = wattn_ref[...]
    a1 = wattn[:, :C]
    a2 = wattn[:, C:]
    e_nc = jax.lax.dot_general(xb_nc, wemb, (((1,), (1,)), ((), ())),
                               preferred_element_type=jnp.float32)
    e_nc = e_nc + beb_row_ref[...]
    s1_t_ref[0] = jax.lax.dot_general(e_nc, a1, (((1,), (1,)), ((), ())),
                                      preferred_element_type=jnp.float32)
    e_cn = jnp.dot(wemb, xb_cn, preferred_element_type=jnp.float32)
    e_cn = e_cn + beb_col_ref[...]
    s2 = jax.lax.dot_general(a2, e_cn, (((1,), (0,)), ((), ())),
                             preferred_element_type=jnp.float32)
    s2_n_ref[0, :, :N] = s2 + battn_ref[0, 0]
    s2_n_ref[0, :, N:] = jnp.zeros_like(s2_n_ref[0, :, N:])
    xp_nc_ref[0, :N, :] = xb_nc.astype(jnp.bfloat16)
    xp_nc_ref[0, N:, :] = jnp.zeros_like(xp_nc_ref[0, N:, :])


def _main_body(xn_nc_ref, xn_cn_ref, x_nc_ref, sq_n_ref,
               s1_t_ref, s2_n_ref, wct_ref, bconv_ref,
               y_ref, sums_ref, sumsq_ref, d_scr, *, blk_r, n_pad):
    b = pl.program_id(0)
    j = pl.program_id(1)
    C = xn_nc_ref.shape[2]

    xr = xn_nc_ref[0]
    xc = xn_cn_ref[0]
    d_scr[...] = sq_n_ref[0] - 2.0 * jnp.dot(
        xr, xc, preferred_element_type=jnp.float32)

    def step(_, m):
        dv = d_scr[...]
        return jnp.min(jnp.where(dv > m, dv, jnp.inf), axis=1, keepdims=True)

    t = jnp.min(d_scr[...], axis=1, keepdims=True)
    t = jax.lax.fori_loop(0, KNN - 1, step, t)

    member = d_scr[...] <= t

    logit = s1_t_ref[0] + s2_n_ref[0]
    logit = jnp.maximum(logit, 0.1 * logit)
    lm = jnp.where(member, logit, jnp.float32(-1e30))
    mx = jnp.max(lm, axis=1, keepdims=True)
    p = jnp.exp(lm - mx)
    w = p * (1.0 / jnp.sum(p, axis=1, keepdims=True))

    x_full = x_nc_ref[0]
    agg = jnp.dot(w.astype(jnp.bfloat16), x_full,
                  preferred_element_type=jnp.float32)
    x_rows = x_nc_ref[0, pl.ds(j * blk_r, blk_r), :]

    wct = wct_ref[...]
    y = (jnp.dot(x_rows, wct[:C], preferred_element_type=jnp.float32)
         + jnp.dot(agg.astype(jnp.bfloat16), wct[C:],
                   preferred_element_type=jnp.float32)
         + bconv_ref[...])
    y_ref[0] = y

    @pl.when(jnp.logical_and(b == 0, j == 0))
    def _():
        sums_ref[...] = jnp.zeros_like(sums_ref)
        sumsq_ref[...] = jnp.zeros_like(sumsq_ref)

    sums_ref[...] += jnp.sum(y, axis=0, keepdims=True)
    sumsq_ref[...] += jnp.sum(y * y, axis=0, keepdims=True)


def _final_body(y_ref, sums_ref, sumsq_ref, gamma_ref, beta_ref, x_nc_ref,
                out_ref, *, count):
    mean = sums_ref[...] / count
    var = sumsq_ref[...] / count - mean * mean
    inv = jax.lax.rsqrt(var + 1e-5)
    y = y_ref[0]
    z = gamma_ref[...] * (y - mean) * inv + beta_ref[...]
    z = jnp.maximum(z, 0.0)
    out_ref[0] = z + x_nc_ref[0]


def kernel(x, W_emb, b_emb, W_attn, b_attn, W_conv, b_conv, gamma, beta):
    B, C, H, W = x.shape
    N = H * W
    NP = ((N + LANES - 1) // LANES) * LANES
    x_cn = x.reshape(B, C, N)
    x_nc = x_cn.transpose(0, 2, 1)

    f32 = jnp.float32
    prep_out = pl.pallas_call(
        _prep_body,
        grid=(B,),
        in_specs=[
            pl.BlockSpec((1, C, N), lambda b: (b, 0, 0)),
            pl.BlockSpec((1, N, C), lambda b: (b, 0, 0)),
            pl.BlockSpec((C, C), lambda b: (0, 0)),
            pl.BlockSpec((1, 2 * C), lambda b: (0, 0)),
            pl.BlockSpec((C, 1), lambda b: (0, 0)),
            pl.BlockSpec((1, C), lambda b: (0, 0)),
            pl.BlockSpec((1, 1), lambda b: (0, 0)),
        ],
        out_specs=[
            pl.BlockSpec((1, C, NP), lambda b: (b, 0, 0)),
            pl.BlockSpec((1, N, C), lambda b: (b, 0, 0)),
            pl.BlockSpec((1, 1, NP), lambda b: (b, 0, 0)),
            pl.BlockSpec((1, N, 1), lambda b: (b, 0, 0)),
            pl.BlockSpec((1, 1, NP), lambda b: (b, 0, 0)),
            pl.BlockSpec((1, NP, C), lambda b: (b, 0, 0)),
        ],
        out_shape=[
            jax.ShapeDtypeStruct((B, C, NP), f32),
            jax.ShapeDtypeStruct((B, N, C), f32),
            jax.ShapeDtypeStruct((B, 1, NP), f32),
            jax.ShapeDtypeStruct((B, N, 1), f32),
            jax.ShapeDtypeStruct((B, 1, NP), f32),
            jax.ShapeDtypeStruct((B, NP, C), jnp.bfloat16),
        ],
    )(x_cn, x_nc, W_emb, W_attn, b_emb.reshape(C, 1), b_emb.reshape(1, C),
      b_attn.reshape(1, 1))
    xn_cn, xn_nc, sq_n, s1_t, s2_n, xp_nc = prep_out

    blk_r = 448
    nb = N // blk_r
    y_nc, sums, sumsq = pl.pallas_call(
        functools.partial(_main_body, blk_r=blk_r, n_pad=NP),
        grid=(B, nb),
        in_specs=[
            pl.BlockSpec((1, blk_r, C), lambda b, j: (b, j, 0)),
            pl.BlockSpec((1, C, NP), lambda b, j: (b, 0, 0)),
            pl.BlockSpec((1, NP, C), lambda b, j: (b, 0, 0)),
            pl.BlockSpec((1, 1, NP), lambda b, j: (b, 0, 0)),
            pl.BlockSpec((1, blk_r, 1), lambda b, j: (b, j, 0)),
            pl.BlockSpec((1, 1, NP), lambda b, j: (b, 0, 0)),
            pl.BlockSpec((2 * C, C), lambda b, j: (0, 0)),
            pl.BlockSpec((1, C), lambda b, j: (0, 0)),
        ],
        out_specs=[
            pl.BlockSpec((1, blk_r, C), lambda b, j: (b, j, 0)),
            pl.BlockSpec((1, C), lambda b, j: (0, 0)),
            pl.BlockSpec((1, C), lambda b, j: (0, 0)),
        ],
        out_shape=[
            jax.ShapeDtypeStruct((B, N, C), f32),
            jax.ShapeDtypeStruct((1, C), f32),
            jax.ShapeDtypeStruct((1, C), f32),
        ],
        scratch_shapes=[pltpu.VMEM((blk_r, NP), f32)],
    )(xn_nc, xn_cn, xp_nc, sq_n, s1_t, s2_n,
      W_conv.T.astype(jnp.bfloat16), b_conv.reshape(1, C))

    out_nc = pl.pallas_call(
        functools.partial(_final_body, count=float(B * N)),
        grid=(B,),
        in_specs=[
            pl.BlockSpec((1, N, C), lambda b: (b, 0, 0)),
            pl.BlockSpec((1, C), lambda b: (0, 0)),
            pl.BlockSpec((1, C), lambda b: (0, 0)),
            pl.BlockSpec((1, C), lambda b: (0, 0)),
            pl.BlockSpec((1, C), lambda b: (0, 0)),
            pl.BlockSpec((1, N, C), lambda b: (b, 0, 0)),
        ],
        out_specs=pl.BlockSpec((1, N, C), lambda b: (b, 0, 0)),
        out_shape=jax.ShapeDtypeStruct((B, N, C), f32),
    )(y_nc, sums, sumsq, gamma.reshape(1, C), beta.reshape(1, C), x_nc)

    return out_nc.transpose(0, 2, 1).reshape(B, C, H, W)


# speedup vs baseline: 1.2347x; 1.0694x over previous
import functools

import jax
import jax.numpy as jnp
from jax.experimental import pallas as pl
from jax.experimental.pallas import tpu as pltpu

KNN = 16
LANES = 128


def _prep_body(xcn_ref, xnc_ref, wemb_ref, wattn_ref, beb_col_ref, beb_row_ref,
               battn_ref, xn_cn_ref, xn_nc_ref, sq_n_ref,
               s1_t_ref, s2_n_ref, xp_nc_ref):
    C = xcn_ref.shape[1]
    N = xcn_ref.shape[2]
    xb_cn = xcn_ref[0]
    xb_nc = xnc_ref[0]
    nrm_col = jnp.sqrt(jnp.sum(xb_cn * xb_cn, axis=1, keepdims=True))
    inv_col = 1.0 / jnp.maximum(nrm_col, 1e-12)
    xn_cn = xb_cn * inv_col
    xn_cn_ref[0, :, :N] = xn_cn
    xn_cn_ref[0, :, N:] = jnp.zeros_like(xn_cn_ref[0, :, N:])
    nrm_row = jnp.sqrt(jnp.sum(xb_nc * xb_nc, axis=0, keepdims=True))
    inv_row = 1.0 / jnp.maximum(nrm_row, 1e-12)
    xn_nc = xb_nc * inv_row
    xn_nc_ref[0] = xn_nc
    sq_n_ref[0, :, :N] = jnp.sum(xn_cn * xn_cn, axis=0, keepdims=True)
    sq_n_ref[0, :, N:] = jnp.full_like(sq_n_ref[0, :, N:], 1e30)
    wemb = wemb_ref[...]
    wattn = wattn_ref[...]
    a1 = wattn[:, :C]
    a2 = wattn[:, C:]
    e_nc = jax.lax.dot_general(xb_nc, wemb, (((1,), (1,)), ((), ())),
                               preferred_element_type=jnp.float32)
    e_nc = e_nc + beb_row_ref[...]
    s1_t_ref[0] = jax.lax.dot_general(e_nc, a1, (((1,), (1,)), ((), ())),
                                      preferred_element_type=jnp.float32)
    e_cn = jnp.dot(wemb, xb_cn, preferred_element_type=jnp.float32)
    e_cn = e_cn + beb_col_ref[...]
    s2 = jax.lax.dot_general(a2, e_cn, (((1,), (0,)), ((), ())),
                             preferred_element_type=jnp.float32)
    s2_n_ref[0, :, :N] = s2 + battn_ref[0, 0]
    s2_n_ref[0, :, N:] = jnp.zeros_like(s2_n_ref[0, :, N:])
    xp_nc_ref[0, :N, :] = xb_nc.astype(jnp.bfloat16)
    xp_nc_ref[0, N:, :] = jnp.zeros_like(xp_nc_ref[0, N:, :])


def _main_body(xn_nc_ref, xn_cn_ref, x_nc_ref, sq_n_ref,
               s1_t_ref, s2_n_ref, wct_ref, bconv_ref,
               y_ref, sums_ref, sumsq_ref, d_scr, *, blk_r, n_pad):
    b = pl.program_id(0)
    j = pl.program_id(1)
    C = xn_nc_ref.shape[2]

    xr = xn_nc_ref[0]
    xc = xn_cn_ref[0]
    d_scr[...] = sq_n_ref[0] - 2.0 * jnp.dot(
        xr, xc, preferred_element_type=jnp.float32)

    def step(_, m):
        dv = d_scr[...]
        return jnp.min(jnp.where(dv > m, dv, jnp.inf), axis=1, keepdims=True)

    t = jnp.min(d_scr[...], axis=1, keepdims=True)
    t = jax.lax.fori_loop(0, KNN - 1, step, t,
                          unroll=5)

    member = d_scr[...] <= t

    logit = s1_t_ref[0] + s2_n_ref[0]
    logit = jnp.maximum(logit, 0.1 * logit)
    lm = jnp.where(member, logit, jnp.float32(-1e30))
    mx = jnp.max(lm, axis=1, keepdims=True)
    p = jnp.exp(lm - mx)
    w = p * (1.0 / jnp.sum(p, axis=1, keepdims=True))

    x_full = x_nc_ref[0]
    agg = jnp.dot(w.astype(jnp.bfloat16), x_full,
                  preferred_element_type=jnp.float32)
    x_rows = x_nc_ref[0, pl.ds(j * blk_r, blk_r), :]

    wct = wct_ref[...]
    y = (jnp.dot(x_rows, wct[:C], preferred_element_type=jnp.float32)
         + jnp.dot(agg.astype(jnp.bfloat16), wct[C:],
                   preferred_element_type=jnp.float32)
         + bconv_ref[...])
    y_ref[0] = y

    @pl.when(jnp.logical_and(b == 0, j == 0))
    def _():
        sums_ref[...] = jnp.zeros_like(sums_ref)
        sumsq_ref[...] = jnp.zeros_like(sumsq_ref)

    sums_ref[...] += jnp.sum(y, axis=0, keepdims=True)
    sumsq_ref[...] += jnp.sum(y * y, axis=0, keepdims=True)


def _final_body(y_ref, sums_ref, sumsq_ref, gamma_ref, beta_ref, x_nc_ref,
                out_ref, *, count):
    mean = sums_ref[...] / count
    var = sumsq_ref[...] / count - mean * mean
    inv = jax.lax.rsqrt(var + 1e-5)
    y = y_ref[0]
    z = gamma_ref[...] * (y - mean) * inv + beta_ref[...]
    z = jnp.maximum(z, 0.0)
    out_ref[0] = z + x_nc_ref[0]


def kernel(x, W_emb, b_emb, W_attn, b_attn, W_conv, b_conv, gamma, beta):
    B, C, H, W = x.shape
    N = H * W
    NP = ((N + LANES - 1) // LANES) * LANES
    x_cn = x.reshape(B, C, N)
    x_nc = x_cn.transpose(0, 2, 1)

    f32 = jnp.float32
    prep_out = pl.pallas_call(
        _prep_body,
        grid=(B,),
        in_specs=[
            pl.BlockSpec((1, C, N), lambda b: (b, 0, 0)),
            pl.BlockSpec((1, N, C), lambda b: (b, 0, 0)),
            pl.BlockSpec((C, C), lambda b: (0, 0)),
            pl.BlockSpec((1, 2 * C), lambda b: (0, 0)),
            pl.BlockSpec((C, 1), lambda b: (0, 0)),
            pl.BlockSpec((1, C), lambda b: (0, 0)),
            pl.BlockSpec((1, 1), lambda b: (0, 0)),
        ],
        out_specs=[
            pl.BlockSpec((1, C, NP), lambda b: (b, 0, 0)),
            pl.BlockSpec((1, N, C), lambda b: (b, 0, 0)),
            pl.BlockSpec((1, 1, NP), lambda b: (b, 0, 0)),
            pl.BlockSpec((1, N, 1), lambda b: (b, 0, 0)),
            pl.BlockSpec((1, 1, NP), lambda b: (b, 0, 0)),
            pl.BlockSpec((1, NP, C), lambda b: (b, 0, 0)),
        ],
        out_shape=[
            jax.ShapeDtypeStruct((B, C, NP), f32),
            jax.ShapeDtypeStruct((B, N, C), f32),
            jax.ShapeDtypeStruct((B, 1, NP), f32),
            jax.ShapeDtypeStruct((B, N, 1), f32),
            jax.ShapeDtypeStruct((B, 1, NP), f32),
            jax.ShapeDtypeStruct((B, NP, C), jnp.bfloat16),
        ],
    )(x_cn, x_nc, W_emb, W_attn, b_emb.reshape(C, 1), b_emb.reshape(1, C),
      b_attn.reshape(1, 1))
    xn_cn, xn_nc, sq_n, s1_t, s2_n, xp_nc = prep_out

    blk_r = 448
    nb = N // blk_r
    y_nc, sums, sumsq = pl.pallas_call(
        functools.partial(_main_body, blk_r=blk_r, n_pad=NP),
        grid=(B, nb),
        in_specs=[
            pl.BlockSpec((1, blk_r, C), lambda b, j: (b, j, 0)),
            pl.BlockSpec((1, C, NP), lambda b, j: (b, 0, 0)),
            pl.BlockSpec((1, NP, C), lambda b, j: (b, 0, 0)),
            pl.BlockSpec((1, 1, NP), lambda b, j: (b, 0, 0)),
            pl.BlockSpec((1, blk_r, 1), lambda b, j: (b, j, 0)),
            pl.BlockSpec((1, 1, NP), lambda b, j: (b, 0, 0)),
            pl.BlockSpec((2 * C, C), lambda b, j: (0, 0)),
            pl.BlockSpec((1, C), lambda b, j: (0, 0)),
        ],
        out_specs=[
            pl.BlockSpec((1, blk_r, C), lambda b, j: (b, j, 0)),
            pl.BlockSpec((1, C), lambda b, j: (0, 0)),
            pl.BlockSpec((1, C), lambda b, j: (0, 0)),
        ],
        out_shape=[
            jax.ShapeDtypeStruct((B, N, C), f32),
            jax.ShapeDtypeStruct((1, C), f32),
            jax.ShapeDtypeStruct((1, C), f32),
        ],
        scratch_shapes=[pltpu.VMEM((blk_r, NP), f32)],
    )(xn_nc, xn_cn, xp_nc, sq_n, s1_t, s2_n,
      W_conv.T.astype(jnp.bfloat16), b_conv.reshape(1, C))

    out_nc = pl.pallas_call(
        functools.partial(_final_body, count=float(B * N)),
        grid=(B,),
        in_specs=[
            pl.BlockSpec((1, N, C), lambda b: (b, 0, 0)),
            pl.BlockSpec((1, C), lambda b: (0, 0)),
            pl.BlockSpec((1, C), lambda b: (0, 0)),
            pl.BlockSpec((1, C), lambda b: (0, 0)),
            pl.BlockSpec((1, C), lambda b: (0, 0)),
            pl.BlockSpec((1, N, C), lambda b: (b, 0, 0)),
        ],
        out_specs=pl.BlockSpec((1, N, C), lambda b: (b, 0, 0)),
        out_shape=jax.ShapeDtypeStruct((B, N, C), f32),
    )(y_nc, sums, sumsq, gamma.reshape(1, C), beta.reshape(1, C), x_nc)

    return out_nc.transpose(0, 2, 1).reshape(B, C, H, W)


# speedup vs baseline: 1.2683x; 1.0271x over previous
import functools

import jax
import jax.numpy as jnp
from jax.experimental import pallas as pl
from jax.experimental.pallas import tpu as pltpu

KNN = 16
LANES = 128


def _prep_body(xcn_ref, xnc_ref, wemb_ref, wattn_ref, beb_col_ref, beb_row_ref,
               battn_ref, xn_cn_ref, xn_nc_ref, sq_n_ref,
               s1_t_ref, s2_n_ref, xp_nc_ref):
    C = xcn_ref.shape[1]
    N = xcn_ref.shape[2]
    xb_cn = xcn_ref[0]
    xb_nc = xnc_ref[0]
    nrm_col = jnp.sqrt(jnp.sum(xb_cn * xb_cn, axis=1, keepdims=True))
    inv_col = 1.0 / jnp.maximum(nrm_col, 1e-12)
    xn_cn = xb_cn * inv_col
    xn_cn_ref[0, :, :N] = xn_cn
    xn_cn_ref[0, :, N:] = jnp.zeros_like(xn_cn_ref[0, :, N:])
    nrm_row = jnp.sqrt(jnp.sum(xb_nc * xb_nc, axis=0, keepdims=True))
    inv_row = 1.0 / jnp.maximum(nrm_row, 1e-12)
    xn_nc = xb_nc * inv_row
    xn_nc_ref[0] = xn_nc
    sq_n_ref[0, :, :N] = jnp.sum(xn_cn * xn_cn, axis=0, keepdims=True)
    sq_n_ref[0, :, N:] = jnp.full_like(sq_n_ref[0, :, N:], 1e30)
    wemb = wemb_ref[...]
    wattn = wattn_ref[...]
    a1 = wattn[:, :C]
    a2 = wattn[:, C:]
    e_nc = jax.lax.dot_general(xb_nc, wemb, (((1,), (1,)), ((), ())),
                               preferred_element_type=jnp.float32)
    e_nc = e_nc + beb_row_ref[...]
    s1_t_ref[0] = jax.lax.dot_general(e_nc, a1, (((1,), (1,)), ((), ())),
                                      preferred_element_type=jnp.float32)
    e_cn = jnp.dot(wemb, xb_cn, preferred_element_type=jnp.float32)
    e_cn = e_cn + beb_col_ref[...]
    s2 = jax.lax.dot_general(a2, e_cn, (((1,), (0,)), ((), ())),
                             preferred_element_type=jnp.float32)
    s2_n_ref[0, :, :N] = s2 + battn_ref[0, 0]
    s2_n_ref[0, :, N:] = jnp.zeros_like(s2_n_ref[0, :, N:])
    xp_nc_ref[0, :N, :] = xb_nc.astype(jnp.bfloat16)
    xp_nc_ref[0, N:, :] = jnp.zeros_like(xp_nc_ref[0, N:, :])


def _main_body(xn_nc_ref, xn_cn_ref, x_nc_ref, sq_n_ref,
               s1_t_ref, s2_n_ref, wct_ref, bconv_ref,
               y_ref, sums_ref, sumsq_ref, d_scr, *, blk_r, n_pad):
    b = pl.program_id(0)
    j = pl.program_id(1)
    C = xn_nc_ref.shape[2]

    xr = xn_nc_ref[0]
    xc = xn_cn_ref[0]
    d_scr[...] = sq_n_ref[0] - 2.0 * jnp.dot(
        xr, xc, preferred_element_type=jnp.float32)

    def step(_, m):
        dv = d_scr[...]
        return jnp.min(jnp.where(dv > m, dv, jnp.inf), axis=1, keepdims=True)

    t = jnp.min(d_scr[...], axis=1, keepdims=True)
    t = jax.lax.fori_loop(0, KNN - 1, step, t,
                          unroll=15)

    member = d_scr[...] <= t

    logit = s1_t_ref[0] + s2_n_ref[0]
    logit = jnp.maximum(logit, 0.1 * logit)
    lm = jnp.where(member, logit, jnp.float32(-1e30))
    mx = jnp.max(lm, axis=1, keepdims=True)
    p = jnp.exp(lm - mx)
    w = p * (1.0 / jnp.sum(p, axis=1, keepdims=True))

    x_full = x_nc_ref[0]
    agg = jnp.dot(w.astype(jnp.bfloat16), x_full,
                  preferred_element_type=jnp.float32)
    x_rows = x_nc_ref[0, pl.ds(j * blk_r, blk_r), :]

    wct = wct_ref[...]
    y = (jnp.dot(x_rows, wct[:C], preferred_element_type=jnp.float32)
         + jnp.dot(agg.astype(jnp.bfloat16), wct[C:],
                   preferred_element_type=jnp.float32)
         + bconv_ref[...])
    y_ref[0] = y

    @pl.when(jnp.logical_and(b == 0, j == 0))
    def _():
        sums_ref[...] = jnp.zeros_like(sums_ref)
        sumsq_ref[...] = jnp.zeros_like(sumsq_ref)

    sums_ref[...] += jnp.sum(y, axis=0, keepdims=True)
    sumsq_ref[...] += jnp.sum(y * y, axis=0, keepdims=True)


def _final_body(y_ref, sums_ref, sumsq_ref, gamma_ref, beta_ref, x_nc_ref,
                out_ref, *, count):
    mean = sums_ref[...] / count
    var = sumsq_ref[...] / count - mean * mean
    inv = jax.lax.rsqrt(var + 1e-5)
    y = y_ref[0]
    z = gamma_ref[...] * (y - mean) * inv + beta_ref[...]
    z = jnp.maximum(z, 0.0)
    out_ref[0] = z + x_nc_ref[0]


def kernel(x, W_emb, b_emb, W_attn, b_attn, W_conv, b_conv, gamma, beta):
    B, C, H, W = x.shape
    N = H * W
    NP = ((N + LANES - 1) // LANES) * LANES
    x_cn = x.reshape(B, C, N)
    x_nc = x_cn.transpose(0, 2, 1)

    f32 = jnp.float32
    prep_out = pl.pallas_call(
        _prep_body,
        grid=(B,),
        in_specs=[
            pl.BlockSpec((1, C, N), lambda b: (b, 0, 0)),
            pl.BlockSpec((1, N, C), lambda b: (b, 0, 0)),
            pl.BlockSpec((C, C), lambda b: (0, 0)),
            pl.BlockSpec((1, 2 * C), lambda b: (0, 0)),
            pl.BlockSpec((C, 1), lambda b: (0, 0)),
            pl.BlockSpec((1, C), lambda b: (0, 0)),
            pl.BlockSpec((1, 1), lambda b: (0, 0)),
        ],
        out_specs=[
            pl.BlockSpec((1, C, NP), lambda b: (b, 0, 0)),
            pl.BlockSpec((1, N, C), lambda b: (b, 0, 0)),
            pl.BlockSpec((1, 1, NP), lambda b: (b, 0, 0)),
            pl.BlockSpec((1, N, 1), lambda b: (b, 0, 0)),
            pl.BlockSpec((1, 1, NP), lambda b: (b, 0, 0)),
            pl.BlockSpec((1, NP, C), lambda b: (b, 0, 0)),
        ],
        out_shape=[
            jax.ShapeDtypeStruct((B, C, NP), f32),
            jax.ShapeDtypeStruct((B, N, C), f32),
            jax.ShapeDtypeStruct((B, 1, NP), f32),
            jax.ShapeDtypeStruct((B, N, 1), f32),
            jax.ShapeDtypeStruct((B, 1, NP), f32),
            jax.ShapeDtypeStruct((B, NP, C), jnp.bfloat16),
        ],
    )(x_cn, x_nc, W_emb, W_attn, b_emb.reshape(C, 1), b_emb.reshape(1, C),
      b_attn.reshape(1, 1))
    xn_cn, xn_nc, sq_n, s1_t, s2_n, xp_nc = prep_out

    blk_r = 448
    nb = N // blk_r
    y_nc, sums, sumsq = pl.pallas_call(
        functools.partial(_main_body, blk_r=blk_r, n_pad=NP),
        grid=(B, nb),
        in_specs=[
            pl.BlockSpec((1, blk_r, C), lambda b, j: (b, j, 0)),
            pl.BlockSpec((1, C, NP), lambda b, j: (b, 0, 0)),
            pl.BlockSpec((1, NP, C), lambda b, j: (b, 0, 0)),
            pl.BlockSpec((1, 1, NP), lambda b, j: (b, 0, 0)),
            pl.BlockSpec((1, blk_r, 1), lambda b, j: (b, j, 0)),
            pl.BlockSpec((1, 1, NP), lambda b, j: (b, 0, 0)),
            pl.BlockSpec((2 * C, C), lambda b, j: (0, 0)),
            pl.BlockSpec((1, C), lambda b, j: (0, 0)),
        ],
        out_specs=[
            pl.BlockSpec((1, blk_r, C), lambda b, j: (b, j, 0)),
            pl.BlockSpec((1, C), lambda b, j: (0, 0)),
            pl.BlockSpec((1, C), lambda b, j: (0, 0)),
        ],
        out_shape=[
            jax.ShapeDtypeStruct((B, N, C), f32),
            jax.ShapeDtypeStruct((1, C), f32),
            jax.ShapeDtypeStruct((1, C), f32),
        ],
        scratch_shapes=[pltpu.VMEM((blk_r, NP), f32)],
    )(xn_nc, xn_cn, xp_nc, sq_n, s1_t, s2_n,
      W_conv.T.astype(jnp.bfloat16), b_conv.reshape(1, C))

    out_nc = pl.pallas_call(
        functools.partial(_final_body, count=float(B * N)),
        grid=(B,),
        in_specs=[
            pl.BlockSpec((1, N, C), lambda b: (b, 0, 0)),
            pl.BlockSpec((1, C), lambda b: (0, 0)),
            pl.BlockSpec((1, C), lambda b: (0, 0)),
            pl.BlockSpec((1, C), lambda b: (0, 0)),
            pl.BlockSpec((1, C), lambda b: (0, 0)),
            pl.BlockSpec((1, N, C), lambda b: (b, 0, 0)),
        ],
        out_specs=pl.BlockSpec((1, N, C), lambda b: (b, 0, 0)),
        out_shape=jax.ShapeDtypeStruct((B, N, C), f32),
    )(y_nc, sums, sumsq, gamma.reshape(1, C), beta.reshape(1, C), x_nc)

    return out_nc.transpose(0, 2, 1).reshape(B, C, H, W)


# speedup vs baseline: 1.2708x; 1.0020x over previous
import functools

import jax
import jax.numpy as jnp
from jax.experimental import pallas as pl
from jax.experimental.pallas import tpu as pltpu

KNN = 16
LANES = 128


def _prep_body(xcn_ref, xnc_ref, wemb_ref, wattn_ref, beb_col_ref, beb_row_ref,
               battn_ref, xn_cn_ref, xn_nc_ref, sq_n_ref,
               s1_t_ref, s2_n_ref, xp_nc_ref):
    C = xcn_ref.shape[1]
    N = xcn_ref.shape[2]
    xb_cn = xcn_ref[0]
    xb_nc = xnc_ref[0]
    nrm_col = jnp.sqrt(jnp.sum(xb_cn * xb_cn, axis=1, keepdims=True))
    inv_col = 1.0 / jnp.maximum(nrm_col, 1e-12)
    xn_cn = xb_cn * inv_col
    xn_cn_ref[0, :, :N] = xn_cn
    xn_cn_ref[0, :, N:] = jnp.zeros_like(xn_cn_ref[0, :, N:])
    nrm_row = jnp.sqrt(jnp.sum(xb_nc * xb_nc, axis=0, keepdims=True))
    inv_row = 1.0 / jnp.maximum(nrm_row, 1e-12)
    xn_nc = xb_nc * inv_row
    xn_nc_ref[0] = xn_nc
    sq_n_ref[0, :, :N] = jnp.sum(xn_cn * xn_cn, axis=0, keepdims=True)
    sq_n_ref[0, :, N:] = jnp.full_like(sq_n_ref[0, :, N:], 1e30)
    wemb = wemb_ref[...]
    wattn = wattn_ref[...]
    a1 = wattn[:, :C]
    a2 = wattn[:, C:]
    e_nc = jax.lax.dot_general(xb_nc, wemb, (((1,), (1,)), ((), ())),
                               preferred_element_type=jnp.float32)
    e_nc = e_nc + beb_row_ref[...]
    s1_t_ref[0] = jax.lax.dot_general(e_nc, a1, (((1,), (1,)), ((), ())),
                                      preferred_element_type=jnp.float32)
    e_cn = jnp.dot(wemb, xb_cn, preferred_element_type=jnp.float32)
    e_cn = e_cn + beb_col_ref[...]
    s2 = jax.lax.dot_general(a2, e_cn, (((1,), (0,)), ((), ())),
                             preferred_element_type=jnp.float32)
    s2_n_ref[0, :, :N] = s2 + battn_ref[0, 0]
    s2_n_ref[0, :, N:] = jnp.zeros_like(s2_n_ref[0, :, N:])
    xp_nc_ref[0, :N, :] = xb_nc.astype(jnp.bfloat16)
    xp_nc_ref[0, N:, :] = jnp.zeros_like(xp_nc_ref[0, N:, :])


def _main_body(xn_nc_ref, xn_cn_ref, x_nc_ref, sq_n_ref,
               s1_t_ref, s2_n_ref, wct_ref, bconv_ref,
               y_ref, sums_ref, sumsq_ref, d_scr, *, blk_r, n_pad):
    b = pl.program_id(0)
    j = pl.program_id(1)
    C = xn_nc_ref.shape[2]

    xr = xn_nc_ref[0]
    xc = xn_cn_ref[0]
    d_scr[...] = sq_n_ref[0] - 2.0 * jnp.dot(
        xr, xc, preferred_element_type=jnp.float32)

    nch = n_pad // LANES

    def masked_min(m):
        parts = []
        for i in range(nch):
            c = d_scr[:, i * LANES:(i + 1) * LANES]
            parts.append(jnp.where(c > m, c, jnp.inf) if m is not None else c)
        while len(parts) > 1:
            nxt = [jnp.minimum(parts[2 * i], parts[2 * i + 1])
                   for i in range(len(parts) // 2)]
            if len(parts) % 2:
                nxt.append(parts[-1])
            parts = nxt
        return jnp.min(parts[0], axis=1, keepdims=True)

    t = jax.lax.fori_loop(0, KNN - 1, lambda _, m: masked_min(m),
                          masked_min(None),
                          unroll=15)

    member = d_scr[...] <= t

    logit = s1_t_ref[0] + s2_n_ref[0]
    logit = jnp.maximum(logit, 0.1 * logit)
    lm = jnp.where(member, logit, jnp.float32(-1e30))
    mx = jnp.max(lm, axis=1, keepdims=True)
    p = jnp.exp(lm - mx)
    w = p * (1.0 / jnp.sum(p, axis=1, keepdims=True))

    x_full = x_nc_ref[0]
    agg = jnp.dot(w.astype(jnp.bfloat16), x_full,
                  preferred_element_type=jnp.float32)
    x_rows = x_nc_ref[0, pl.ds(j * blk_r, blk_r), :]

    wct = wct_ref[...]
    y = (jnp.dot(x_rows, wct[:C], preferred_element_type=jnp.float32)
         + jnp.dot(agg.astype(jnp.bfloat16), wct[C:],
                   preferred_element_type=jnp.float32)
         + bconv_ref[...])
    y_ref[0] = y

    @pl.when(jnp.logical_and(b == 0, j == 0))
    def _():
        sums_ref[...] = jnp.zeros_like(sums_ref)
        sumsq_ref[...] = jnp.zeros_like(sumsq_ref)

    sums_ref[...] += jnp.sum(y, axis=0, keepdims=True)
    sumsq_ref[...] += jnp.sum(y * y, axis=0, keepdims=True)


def _final_body(y_ref, sums_ref, sumsq_ref, gamma_ref, beta_ref, x_nc_ref,
                out_ref, *, count):
    mean = sums_ref[...] / count
    var = sumsq_ref[...] / count - mean * mean
    inv = jax.lax.rsqrt(var + 1e-5)
    y = y_ref[0]
    z = gamma_ref[...] * (y - mean) * inv + beta_ref[...]
    z = jnp.maximum(z, 0.0)
    out_ref[0] = z + x_nc_ref[0]


def kernel(x, W_emb, b_emb, W_attn, b_attn, W_conv, b_conv, gamma, beta):
    B, C, H, W = x.shape
    N = H * W
    NP = ((N + LANES - 1) // LANES) * LANES
    x_cn = x.reshape(B, C, N)
    x_nc = x_cn.transpose(0, 2, 1)

    f32 = jnp.float32
    prep_out = pl.pallas_call(
        _prep_body,
        grid=(B,),
        in_specs=[
            pl.BlockSpec((1, C, N), lambda b: (b, 0, 0)),
            pl.BlockSpec((1, N, C), lambda b: (b, 0, 0)),
            pl.BlockSpec((C, C), lambda b: (0, 0)),
            pl.BlockSpec((1, 2 * C), lambda b: (0, 0)),
            pl.BlockSpec((C, 1), lambda b: (0, 0)),
            pl.BlockSpec((1, C), lambda b: (0, 0)),
            pl.BlockSpec((1, 1), lambda b: (0, 0)),
        ],
        out_specs=[
            pl.BlockSpec((1, C, NP), lambda b: (b, 0, 0)),
            pl.BlockSpec((1, N, C), lambda b: (b, 0, 0)),
            pl.BlockSpec((1, 1, NP), lambda b: (b, 0, 0)),
            pl.BlockSpec((1, N, 1), lambda b: (b, 0, 0)),
            pl.BlockSpec((1, 1, NP), lambda b: (b, 0, 0)),
            pl.BlockSpec((1, NP, C), lambda b: (b, 0, 0)),
        ],
        out_shape=[
            jax.ShapeDtypeStruct((B, C, NP), f32),
            jax.ShapeDtypeStruct((B, N, C), f32),
            jax.ShapeDtypeStruct((B, 1, NP), f32),
            jax.ShapeDtypeStruct((B, N, 1), f32),
            jax.ShapeDtypeStruct((B, 1, NP), f32),
            jax.ShapeDtypeStruct((B, NP, C), jnp.bfloat16),
        ],
    )(x_cn, x_nc, W_emb, W_attn, b_emb.reshape(C, 1), b_emb.reshape(1, C),
      b_attn.reshape(1, 1))
    xn_cn, xn_nc, sq_n, s1_t, s2_n, xp_nc = prep_out

    blk_r = 448
    nb = N // blk_r
    y_nc, sums, sumsq = pl.pallas_call(
        functools.partial(_main_body, blk_r=blk_r, n_pad=NP),
        grid=(B, nb),
        in_specs=[
            pl.BlockSpec((1, blk_r, C), lambda b, j: (b, j, 0)),
            pl.BlockSpec((1, C, NP), lambda b, j: (b, 0, 0)),
            pl.BlockSpec((1, NP, C), lambda b, j: (b, 0, 0)),
            pl.BlockSpec((1, 1, NP), lambda b, j: (b, 0, 0)),
            pl.BlockSpec((1, blk_r, 1), lambda b, j: (b, j, 0)),
            pl.BlockSpec((1, 1, NP), lambda b, j: (b, 0, 0)),
            pl.BlockSpec((2 * C, C), lambda b, j: (0, 0)),
            pl.BlockSpec((1, C), lambda b, j: (0, 0)),
        ],
        out_specs=[
            pl.BlockSpec((1, blk_r, C), lambda b, j: (b, j, 0)),
            pl.BlockSpec((1, C), lambda b, j: (0, 0)),
            pl.BlockSpec((1, C), lambda b, j: (0, 0)),
        ],
        out_shape=[
            jax.ShapeDtypeStruct((B, N, C), f32),
            jax.ShapeDtypeStruct((1, C), f32),
            jax.ShapeDtypeStruct((1, C), f32),
        ],
        scratch_shapes=[pltpu.VMEM((blk_r, NP), f32)],
    )(xn_nc, xn_cn, xp_nc, sq_n, s1_t, s2_n,
      W_conv.T.astype(jnp.bfloat16), b_conv.reshape(1, C))

    out_nc = pl.pallas_call(
        functools.partial(_final_body, count=float(B * N)),
        grid=(B,),
        in_specs=[
            pl.BlockSpec((1, N, C), lambda b: (b, 0, 0)),
            pl.BlockSpec((1, C), lambda b: (0, 0)),
            pl.BlockSpec((1, C), lambda b: (0, 0)),
            pl.BlockSpec((1, C), lambda b: (0, 0)),
            pl.BlockSpec((1, C), lambda b: (0, 0)),
            pl.BlockSpec((1, N, C), lambda b: (b, 0, 0)),
        ],
        out_specs=pl.BlockSpec((1, N, C), lambda b: (b, 0, 0)),
        out_shape=jax.ShapeDtypeStruct((B, N, C), f32),
    )(y_nc, sums, sumsq, gamma.reshape(1, C), beta.reshape(1, C), x_nc)

    return out_nc.transpose(0, 2, 1).reshape(B, C, H, W)


# speedup vs baseline: 1.2724x; 1.0013x over previous
import functools

import jax
import jax.numpy as jnp
from jax.experimental import pallas as pl
from jax.experimental.pallas import tpu as pltpu

KNN = 16
LANES = 128


def _prep_body(xcn_ref, xnc_ref, wemb_ref, wattn_ref, beb_col_ref, beb_row_ref,
               battn_ref, xn_cn_ref, xn_nc_ref, sq_n_ref,
               s1_t_ref, s2_n_ref, xp_nc_ref):
    C = xcn_ref.shape[1]
    N = xcn_ref.shape[2]
    xb_cn = xcn_ref[0]
    xb_nc = xnc_ref[0]
    nrm_col = jnp.sqrt(jnp.sum(xb_cn * xb_cn, axis=1, keepdims=True))
    inv_col = 1.0 / jnp.maximum(nrm_col, 1e-12)
    xn_cn = xb_cn * inv_col
    xn_cn_ref[0, :, :N] = xn_cn
    xn_cn_ref[0, :, N:] = jnp.zeros_like(xn_cn_ref[0, :, N:])
    nrm_row = jnp.sqrt(jnp.sum(xb_nc * xb_nc, axis=0, keepdims=True))
    inv_row = 1.0 / jnp.maximum(nrm_row, 1e-12)
    xn_nc = xb_nc * inv_row
    xn_nc_ref[0] = xn_nc
    sq_n_ref[0, :, :N] = jnp.sum(xn_cn * xn_cn, axis=0, keepdims=True)
    sq_n_ref[0, :, N:] = jnp.full_like(sq_n_ref[0, :, N:], 1e30)
    wemb = wemb_ref[...]
    wattn = wattn_ref[...]
    a1 = wattn[:, :C]
    a2 = wattn[:, C:]
    e_nc = jax.lax.dot_general(xb_nc, wemb, (((1,), (1,)), ((), ())),
                               preferred_element_type=jnp.float32)
    e_nc = e_nc + beb_row_ref[...]
    s1_t_ref[0] = jax.lax.dot_general(e_nc, a1, (((1,), (1,)), ((), ())),
                                      preferred_element_type=jnp.float32)
    e_cn = jnp.dot(wemb, xb_cn, preferred_element_type=jnp.float32)
    e_cn = e_cn + beb_col_ref[...]
    s2 = jax.lax.dot_general(a2, e_cn, (((1,), (0,)), ((), ())),
                             preferred_element_type=jnp.float32)
    s2_n_ref[0, :, :N] = s2 + battn_ref[0, 0]
    s2_n_ref[0, :, N:] = jnp.zeros_like(s2_n_ref[0, :, N:])
    xp_nc_ref[0, :N, :] = xb_nc.astype(jnp.bfloat16)
    xp_nc_ref[0, N:, :] = jnp.zeros_like(xp_nc_ref[0, N:, :])


def _main_body(xn_nc_ref, xn_cn_ref, x_nc_ref, sq_n_ref,
               s1_t_ref, s2_n_ref, wct_ref, bconv_ref,
               y_ref, sums_ref, sumsq_ref, d_scr, *, blk_r, n_pad):
    b = pl.program_id(0)
    j = pl.program_id(1)
    C = xn_nc_ref.shape[2]

    xr = xn_nc_ref[0]
    xc = xn_cn_ref[0]
    d_scr[...] = sq_n_ref[0] - 2.0 * jnp.dot(
        xr, xc, preferred_element_type=jnp.float32)

    nch = n_pad // LANES

    def masked_min(m):
        parts = []
        for i in range(nch):
            c = d_scr[:, i * LANES:(i + 1) * LANES]
            parts.append(jnp.where(c > m, c, jnp.inf) if m is not None else c)
        while len(parts) > 1:
            nxt = [jnp.minimum(parts[2 * i], parts[2 * i + 1])
                   for i in range(len(parts) // 2)]
            if len(parts) % 2:
                nxt.append(parts[-1])
            parts = nxt
        return jnp.min(parts[0], axis=1, keepdims=True)

    t = jax.lax.fori_loop(0, KNN - 1, lambda _, m: masked_min(m),
                          masked_min(None),
                          unroll=15)

    member = d_scr[...] <= t

    logit = s1_t_ref[0] + s2_n_ref[0]
    logit = jnp.maximum(logit, 0.1 * logit)
    lm = jnp.where(member, logit, jnp.float32(-1e30))
    mx = jnp.max(lm, axis=1, keepdims=True)
    p = jnp.exp(lm - mx)
    w = p * (1.0 / jnp.sum(p, axis=1, keepdims=True))

    x_full = x_nc_ref[0]
    agg = jnp.dot(w.astype(jnp.bfloat16), x_full,
                  preferred_element_type=jnp.float32)
    x_rows = x_nc_ref[0, pl.ds(j * blk_r, blk_r), :]

    wct = wct_ref[...]
    y = (jnp.dot(x_rows, wct[:C], preferred_element_type=jnp.float32)
         + jnp.dot(agg.astype(jnp.bfloat16), wct[C:],
                   preferred_element_type=jnp.float32)
         + bconv_ref[...])
    y_ref[0] = y

    @pl.when(jnp.logical_and(b == 0, j == 0))
    def _():
        sums_ref[...] = jnp.zeros_like(sums_ref)
        sumsq_ref[...] = jnp.zeros_like(sumsq_ref)

    sums_ref[...] += jnp.sum(y, axis=0, keepdims=True)
    sumsq_ref[...] += jnp.sum(y * y, axis=0, keepdims=True)


def _final_body(y_ref, sums_ref, sumsq_ref, gamma_ref, beta_ref, x_nc_ref,
                out_ref, *, count):
    mean = sums_ref[...] / count
    var = sumsq_ref[...] / count - mean * mean
    inv = jax.lax.rsqrt(var + 1e-5)
    y = y_ref[0]
    z = gamma_ref[...] * (y - mean) * inv + beta_ref[...]
    z = jnp.maximum(z, 0.0)
    out_ref[0] = z + x_nc_ref[0]


def kernel(x, W_emb, b_emb, W_attn, b_attn, W_conv, b_conv, gamma, beta):
    B, C, H, W = x.shape
    N = H * W
    NP = ((N + LANES - 1) // LANES) * LANES
    x_cn = x.reshape(B, C, N)
    x_nc = x_cn.transpose(0, 2, 1)

    f32 = jnp.float32
    prep_out = pl.pallas_call(
        _prep_body,
        grid=(B,),
        in_specs=[
            pl.BlockSpec((1, C, N), lambda b: (b, 0, 0)),
            pl.BlockSpec((1, N, C), lambda b: (b, 0, 0)),
            pl.BlockSpec((C, C), lambda b: (0, 0)),
            pl.BlockSpec((1, 2 * C), lambda b: (0, 0)),
            pl.BlockSpec((C, 1), lambda b: (0, 0)),
            pl.BlockSpec((1, C), lambda b: (0, 0)),
            pl.BlockSpec((1, 1), lambda b: (0, 0)),
        ],
        out_specs=[
            pl.BlockSpec((1, C, NP), lambda b: (b, 0, 0)),
            pl.BlockSpec((1, N, C), lambda b: (b, 0, 0)),
            pl.BlockSpec((1, 1, NP), lambda b: (b, 0, 0)),
            pl.BlockSpec((1, N, 1), lambda b: (b, 0, 0)),
            pl.BlockSpec((1, 1, NP), lambda b: (b, 0, 0)),
            pl.BlockSpec((1, NP, C), lambda b: (b, 0, 0)),
        ],
        out_shape=[
            jax.ShapeDtypeStruct((B, C, NP), f32),
            jax.ShapeDtypeStruct((B, N, C), f32),
            jax.ShapeDtypeStruct((B, 1, NP), f32),
            jax.ShapeDtypeStruct((B, N, 1), f32),
            jax.ShapeDtypeStruct((B, 1, NP), f32),
            jax.ShapeDtypeStruct((B, NP, C), jnp.bfloat16),
        ],
    )(x_cn, x_nc, W_emb, W_attn, b_emb.reshape(C, 1), b_emb.reshape(1, C),
      b_attn.reshape(1, 1))
    xn_cn, xn_nc, sq_n, s1_t, s2_n, xp_nc = prep_out

    blk_r = 784
    nb = N // blk_r
    y_nc, sums, sumsq = pl.pallas_call(
        functools.partial(_main_body, blk_r=blk_r, n_pad=NP),
        grid=(B, nb),
        in_specs=[
            pl.BlockSpec((1, blk_r, C), lambda b, j: (b, j, 0)),
            pl.BlockSpec((1, C, NP), lambda b, j: (b, 0, 0)),
            pl.BlockSpec((1, NP, C), lambda b, j: (b, 0, 0)),
            pl.BlockSpec((1, 1, NP), lambda b, j: (b, 0, 0)),
            pl.BlockSpec((1, blk_r, 1), lambda b, j: (b, j, 0)),
            pl.BlockSpec((1, 1, NP), lambda b, j: (b, 0, 0)),
            pl.BlockSpec((2 * C, C), lambda b, j: (0, 0)),
            pl.BlockSpec((1, C), lambda b, j: (0, 0)),
        ],
        out_specs=[
            pl.BlockSpec((1, blk_r, C), lambda b, j: (b, j, 0)),
            pl.BlockSpec((1, C), lambda b, j: (0, 0)),
            pl.BlockSpec((1, C), lambda b, j: (0, 0)),
        ],
        out_shape=[
            jax.ShapeDtypeStruct((B, N, C), f32),
            jax.ShapeDtypeStruct((1, C), f32),
            jax.ShapeDtypeStruct((1, C), f32),
        ],
        scratch_shapes=[pltpu.VMEM((blk_r, NP), f32)],
    )(xn_nc, xn_cn, xp_nc, sq_n, s1_t, s2_n,
      W_conv.T.astype(jnp.bfloat16), b_conv.reshape(1, C))

    out_nc = pl.pallas_call(
        functools.partial(_final_body, count=float(B * N)),
        grid=(B,),
        in_specs=[
            pl.BlockSpec((1, N, C), lambda b: (b, 0, 0)),
            pl.BlockSpec((1, C), lambda b: (0, 0)),
            pl.BlockSpec((1, C), lambda b: (0, 0)),
            pl.BlockSpec((1, C), lambda b: (0, 0)),
            pl.BlockSpec((1, C), lambda b: (0, 0)),
            pl.BlockSpec((1, N, C), lambda b: (b, 0, 0)),
        ],
        out_specs=pl.BlockSpec((1, N, C), lambda b: (b, 0, 0)),
        out_shape=jax.ShapeDtypeStruct((B, N, C), f32),
    )(y_nc, sums, sumsq, gamma.reshape(1, C), beta.reshape(1, C), x_nc)

    return out_nc.transpose(0, 2, 1).reshape(B, C, H, W)
